# baseline (speedup 1.0000x reference)
import functools
import math

import jax
import jax.numpy as jnp
from jax import lax
from jax.experimental import pallas as pl
from jax.experimental.pallas import tpu as pltpu
from jax.experimental.pallas import tpu_sc as plsc

D_MODEL = 64
SCALE = math.sqrt(D_MODEL)
NUM_CORES = 2
NUM_SUBCORES = 16
NUM_WORKERS = NUM_CORES * NUM_SUBCORES
SUB = 128
NBUF = 3
LANES = 16


def _emb_body(x_hbm, tq_hbm, out_hbm, idx_v, qbuf_v, rows_v, tile_v,
              g0, g1, g2, o0, o1, o2):
    gsems = (g0, g1, g2)
    osems = (o0, o1, o2)

    n_j, n_k, n_i = out_hbm.shape
    n_ib = n_i // SUB
    n_units_total = n_j * n_ib
    n_units = n_units_total // NUM_WORKERS

    wid = lax.axis_index("s") * NUM_CORES + lax.axis_index("c")
    u_base = wid * n_units

    iota = lax.iota(jnp.int32, LANES)

    def prep_gather(u_local, b):
        def halve(c16, carry):
            qbuf_v[b, pl.ds(c16 * LANES, LANES)] = lax.shift_right_logical(
                idx_v[u_local, pl.ds(c16 * LANES, LANES)], 1
            )
            return carry
        lax.fori_loop(0, SUB // LANES, halve, 0, unroll=8)
        pltpu.async_copy(
            tq_hbm.at[qbuf_v.at[b]], rows_v.at[b], gsems[b]
        )

    def drain_gather(b):
        pltpu.make_async_copy(
            tq_hbm.at[qbuf_v.at[0]], rows_v.at[b], gsems[b]
        ).wait()

    def transpose_scale(u_local, b):
        for t in range(SUB // LANES):
            rows16 = iota + (t * LANES)
            idxvals = idx_v[u_local, pl.ds(t * LANES, LANES)]
            half = lax.shift_left(jnp.bitwise_and(idxvals, 1), 6)

            @plsc.parallel_loop(0, D_MODEL, 4, unroll=4)
            def col_body(c):
                for dc in range(4):
                    v = plsc.load_gather(rows_v.at[b], [rows16, half + (c + dc)])
                    tile_v[b, c + dc, pl.ds(t * LANES, LANES)] = v * SCALE

    def fire_out(u, b):
        j = u // n_ib
        ib = u - j * n_ib
        pltpu.async_copy(
            tile_v.at[b],
            out_hbm.at[j, :, pl.ds(ib * SUB, SUB)],
            osems[b],
        )

    def wait_out(b):
        pltpu.make_async_copy(
            tile_v.at[b], out_hbm.at[0, :, pl.ds(0, SUB)], osems[b]
        ).wait()

    pltpu.sync_copy(x_hbm.at[pl.ds(u_base, n_units)], idx_v)

    prep_gather(0, 0)
    prep_gather(1, 1)
    drain_gather(0)
    transpose_scale(0, 0)
    fire_out(u_base + 0, 0)
    prep_gather(2, 2)
    drain_gather(1)
    transpose_scale(1, 1)
    fire_out(u_base + 1, 1)
    prep_gather(3, 0)
    drain_gather(2)
    transpose_scale(2, 2)
    fire_out(u_base + 2, 2)
    prep_gather(4, 1)

    def slot(ul, b):
        drain_gather(b)
        wait_out(b)
        transpose_scale(ul, b)
        fire_out(u_base + ul, b)
        prep_gather(ul + 2, (b + 2) % NBUF)

    def group(h, carry):
        ul = 3 + h * NBUF
        slot(ul, 0)
        slot(ul + 1, 1)
        slot(ul + 2, 2)
        return carry

    n_groups = (n_units - 5) // NBUF
    lax.fori_loop(0, n_groups, group, 0)

    ul = 3 + n_groups * NBUF
    drain_gather(0)
    wait_out(0)
    transpose_scale(ul, 0)
    fire_out(u_base + ul, 0)
    drain_gather(1)
    wait_out(1)
    transpose_scale(ul + 1, 1)
    fire_out(u_base + ul + 1, 1)
    wait_out(2)
    wait_out(0)
    wait_out(1)


def kernel(x, table):
    b0, b1 = x.shape
    v_rows = table.shape[0]
    tq = table.reshape(v_rows // 2, 2 * D_MODEL)
    x_units = jnp.transpose(x, (1, 0)).reshape(b1 * (b0 // SUB), SUB)
    x_units = x_units.astype(jnp.int32)

    mesh = plsc.VectorSubcoreMesh(
        core_axis_name="c",
        subcore_axis_name="s",
        num_cores=NUM_CORES,
        num_subcores=NUM_SUBCORES,
    )
    emb = pl.kernel(
        _emb_body,
        out_type=jax.ShapeDtypeStruct((b1, D_MODEL, b0), jnp.float32),
        mesh=mesh,
        scratch_types=[
            pltpu.VMEM((b1 * (b0 // SUB) // NUM_WORKERS, SUB), jnp.int32),
            pltpu.VMEM((NBUF, SUB), jnp.int32),
            pltpu.VMEM((NBUF, SUB, 2 * D_MODEL), jnp.float32),
            pltpu.VMEM((NBUF, D_MODEL, SUB), jnp.float32),
            pltpu.SemaphoreType.DMA,
            pltpu.SemaphoreType.DMA,
            pltpu.SemaphoreType.DMA,
            pltpu.SemaphoreType.DMA,
            pltpu.SemaphoreType.DMA,
            pltpu.SemaphoreType.DMA,
        ],
        compiler_params=pltpu.CompilerParams(
            use_tc_tiling_on_sc=True, needs_layout_passes=False
        ),
    )
    out_t = emb(x_units, tq)
    return jnp.transpose(out_t, (2, 0, 1))


# speedup vs baseline: 1.0501x; 1.0501x over previous
import functools
import math

import jax
import jax.numpy as jnp
from jax import lax
from jax.experimental import pallas as pl
from jax.experimental.pallas import tpu as pltpu
from jax.experimental.pallas import tpu_sc as plsc

D_MODEL = 64
SCALE = math.sqrt(D_MODEL)
NUM_CORES = 2
NUM_SUBCORES = 16
NUM_WORKERS = NUM_CORES * NUM_SUBCORES
SUB = 128
CHUNK = 512
SUBS_PER_CHUNK = CHUNK // SUB
NBUF = 3
LANES = 16


def _emb_body(x_hbm, table_hbm, out_hbm, idx_v, rows_v,
              g0, g1, g2, o0, o1, o2):
    gsems = (g0, g1, g2)
    osems = (o0, o1, o2)

    b_total = out_hbm.shape[0]
    b_per_w = b_total // NUM_WORKERS
    n_chunks = b_per_w // CHUNK
    idx_rows = b_per_w // SUB

    wid = lax.axis_index("s") * NUM_CORES + lax.axis_index("c")
    base_row = wid * b_per_w
    base_idx_row = wid * idx_rows

    def fire_gathers(c, b):
        for j in range(SUBS_PER_CHUNK):
            pltpu.async_copy(
                table_hbm.at[idx_v.at[c * SUBS_PER_CHUNK + j]],
                rows_v.at[b].at[pl.ds(j * SUB, SUB)],
                gsems[b],
            )

    def drain_gathers(b):
        pltpu.make_async_copy(
            table_hbm.at[idx_v.at[0]], rows_v.at[b], gsems[b]
        ).wait()

    def scale(b):
        @plsc.parallel_loop(0, CHUNK, 1, unroll=8)
        def scale_body(r):
            for cc in range(D_MODEL // LANES):
                rows_v[b, r, pl.ds(cc * LANES, LANES)] = (
                    rows_v[b, r, pl.ds(cc * LANES, LANES)] * SCALE
                )

    def fire_out(c, b):
        pltpu.async_copy(
            rows_v.at[b],
            out_hbm.at[pl.ds(base_row + c * CHUNK, CHUNK)],
            osems[b],
        )

    def wait_out(b):
        pltpu.make_async_copy(
            rows_v.at[b], out_hbm.at[pl.ds(0, CHUNK)], osems[b]
        ).wait()

    pltpu.sync_copy(x_hbm.at[pl.ds(base_idx_row, idx_rows)], idx_v)

    fire_gathers(0, 0)
    fire_gathers(1, 1)
    drain_gathers(0)
    scale(0)
    fire_out(0, 0)
    fire_gathers(2, 2)
    drain_gathers(1)
    scale(1)
    fire_out(1, 1)
    wait_out(0)
    fire_gathers(3, 0)

    def slot(c, b):
        drain_gathers(b)
        scale(b)
        fire_out(c, b)
        wait_out((b + 2) % NBUF)
        fire_gathers(c + 2, (b + 2) % NBUF)

    def group(h, carry):
        c = 2 + h * NBUF
        slot(c, 2)
        slot(c + 1, 0)
        slot(c + 2, 1)
        return carry

    n_groups = (n_chunks - 2 - NBUF) // NBUF
    lax.fori_loop(0, n_groups, group, 0)

    c_tail = 2 + n_groups * NBUF
    drain_gathers(2)
    scale(2)
    fire_out(c_tail, 2)
    wait_out(1)
    fire_gathers(c_tail + 2, 1)
    drain_gathers(0)
    scale(0)
    fire_out(c_tail + 1, 0)
    drain_gathers(1)
    scale(1)
    fire_out(c_tail + 2, 1)
    for b in range(NBUF):
        wait_out(b)


def kernel(x, table):
    b0, b1 = x.shape
    b_total = b0 * b1
    x2d = x.reshape(b_total // SUB, SUB).astype(jnp.int32)

    mesh = plsc.VectorSubcoreMesh(
        core_axis_name="c",
        subcore_axis_name="s",
        num_cores=NUM_CORES,
        num_subcores=NUM_SUBCORES,
    )
    emb = pl.kernel(
        _emb_body,
        out_type=jax.ShapeDtypeStruct((b_total, D_MODEL), jnp.float32),
        mesh=mesh,
        scratch_types=[
            pltpu.VMEM((b_total // NUM_WORKERS // SUB, SUB), jnp.int32),
            pltpu.VMEM((NBUF, CHUNK, D_MODEL), jnp.float32),
            pltpu.SemaphoreType.DMA,
            pltpu.SemaphoreType.DMA,
            pltpu.SemaphoreType.DMA,
            pltpu.SemaphoreType.DMA,
            pltpu.SemaphoreType.DMA,
            pltpu.SemaphoreType.DMA,
        ],
        compiler_params=pltpu.CompilerParams(use_tc_tiling_on_sc=False),
    )
    out = emb(x2d, table)
    return out.reshape(b0, b1, D_MODEL)
